# baseline (speedup 1.0000x reference)
import functools

import jax
import jax.numpy as jnp
from jax import lax
from jax.experimental import pallas as pl
from jax.experimental.pallas import tpu as pltpu
from jax.experimental.pallas import tpu_sc as plsc

D_MODEL = 64
VOCAB = 1_000_000

VB = 2048
NBLK = -(-VOCAB // VB)
VPAD = NBLK * VB


def _transpose_body(w_ref, out_ref):
    out_ref[...] = w_ref[...].T


def _transpose_table(W_E):
    return pl.pallas_call(
        _transpose_body,
        grid=(NBLK,),
        in_specs=[pl.BlockSpec((D_MODEL, VB), lambda i: (0, i))],
        out_specs=pl.BlockSpec((VB, D_MODEL), lambda i: (i, 0)),
        out_shape=jax.ShapeDtypeStruct((VPAD, D_MODEL), jnp.float32),
    )(W_E)


B_TOTAL = 4096 * 200
NC, NS = 2, 16
NW = NC * NS
B_PER_W = B_TOTAL // NW
CH = 128
NCHUNK = B_PER_W // CH


def _sc_gather(W_T, idx):
    mesh = plsc.VectorSubcoreMesh(core_axis_name="c", subcore_axis_name="s")

    @functools.partial(
        pl.kernel,
        mesh=mesh,
        out_type=jax.ShapeDtypeStruct((B_TOTAL, D_MODEL), jnp.float32),
        scratch_types=[
            pltpu.VMEM((NCHUNK, CH), jnp.int32),
            pltpu.VMEM((CH, D_MODEL), jnp.float32),
            pltpu.SemaphoreType.DMA,
        ],
        compiler_params=pltpu.CompilerParams(use_tc_tiling_on_sc=False),
    )
    def k(table_hbm, idx_hbm, out_hbm, idx_v, rows_v, sem):
        wid = lax.axis_index("s") * NC + lax.axis_index("c")
        base = wid * B_PER_W
        pltpu.sync_copy(idx_hbm.at[wid], idx_v)

        def body(j, carry):
            pltpu.async_copy(table_hbm.at[idx_v.at[j]], rows_v, sem).wait()
            pltpu.sync_copy(rows_v, out_hbm.at[pl.ds(base + j * CH, CH)])
            return carry

        lax.fori_loop(0, NCHUNK, body, 0, unroll=False)

    return k(W_T, idx)


def kernel(x, W_E):
    W_T = _transpose_table(W_E)
    idx = x.reshape(NW, NCHUNK, CH)
    out = _sc_gather(W_T, idx)
    return out.reshape(4096, 200, D_MODEL)


# speedup vs baseline: 1.5418x; 1.5418x over previous
import functools

import jax
import jax.numpy as jnp
from jax import lax
from jax.experimental import pallas as pl
from jax.experimental.pallas import tpu as pltpu
from jax.experimental.pallas import tpu_sc as plsc

D_MODEL = 64
VOCAB = 1_000_000
B = 4096
P = 200

VB = 2048
NBLK = -(-((VOCAB + 1) // 2) // VB)
H = NBLK * VB


def _pack_body(wl_ref, wr_ref, out_ref):
    out_ref[:, 0:64] = wl_ref[...].T
    out_ref[:, 64:128] = wr_ref[...].T


def _pack_table(W_E):
    return pl.pallas_call(
        _pack_body,
        grid=(NBLK,),
        in_specs=[
            pl.BlockSpec((D_MODEL, VB), lambda i: (0, i)),
            pl.BlockSpec(
                (D_MODEL, VB),
                lambda i: (0, jnp.minimum(i + NBLK, (VOCAB - 1) // VB)),
            ),
        ],
        out_specs=pl.BlockSpec((VB, 2 * D_MODEL), lambda i: (i, 0)),
        out_shape=jax.ShapeDtypeStruct((H, 2 * D_MODEL), jnp.float32),
    )(W_E, W_E)


B_TOTAL = B * P
NC, NS = 2, 16
NW = NC * NS
B_PER_W = B_TOTAL // NW
CH = 128
NCHUNK = B_PER_W // CH


def _sc_gather(table, idx):
    mesh = plsc.VectorSubcoreMesh(core_axis_name="c", subcore_axis_name="s")

    @functools.partial(
        pl.kernel,
        mesh=mesh,
        out_type=jax.ShapeDtypeStruct((B_TOTAL, D_MODEL), jnp.float32),
        scratch_types=[
            pltpu.VMEM((NCHUNK, CH), jnp.int32),
            pltpu.VMEM((CH, D_MODEL), jnp.float32),
            pltpu.SemaphoreType.DMA,
        ],
        compiler_params=pltpu.CompilerParams(use_tc_tiling_on_sc=False),
    )
    def k(table_hbm, idx_hbm, out_hbm, idx_v, rows_v, sem):
        wid = lax.axis_index("s") * NC + lax.axis_index("c")
        base = wid * B_PER_W
        pltpu.sync_copy(idx_hbm.at[wid], idx_v)

        def body(j, carry):
            pltpu.async_copy(table_hbm.at[idx_v.at[j]], rows_v, sem).wait()
            pltpu.sync_copy(rows_v, out_hbm.at[pl.ds(base + j * CH, CH)])
            return carry

        lax.fori_loop(0, NCHUNK, body, 0, unroll=False)

    return k(table, idx)


B1 = 128
NB3 = B // B1


def _relayout_body(in_ref, out_ref):
    for p2 in range(P // 2):
        x2 = in_ref[:, p2, :]
        out_ref[2 * p2] = x2[:, 0:64].T
        out_ref[2 * p2 + 1] = x2[:, 64:128].T


def _relayout(out_lin):
    v = out_lin.reshape(B, P // 2, 2 * D_MODEL)
    return pl.pallas_call(
        _relayout_body,
        grid=(NB3,),
        in_specs=[pl.BlockSpec((B1, P // 2, 2 * D_MODEL), lambda i: (i, 0, 0))],
        out_specs=pl.BlockSpec((P, D_MODEL, B1), lambda i: (0, 0, i)),
        out_shape=jax.ShapeDtypeStruct((P, D_MODEL, B), jnp.float32),
    )(v)


def kernel(x, W_E):
    table2 = _pack_table(W_E)
    table = table2.reshape(2 * H, D_MODEL)
    xi = x.astype(jnp.int32)
    j = jnp.where(xi < H, 2 * xi, 2 * (xi - H) + 1)
    idx = j.reshape(NW, NCHUNK, CH)
    out_lin = _sc_gather(table, idx)
    g = _relayout(out_lin)
    return jnp.transpose(g, (2, 0, 1))


# speedup vs baseline: 1.6227x; 1.0525x over previous
import functools

import jax
import jax.numpy as jnp
from jax import lax
from jax.experimental import pallas as pl
from jax.experimental.pallas import tpu as pltpu
from jax.experimental.pallas import tpu_sc as plsc

D_MODEL = 64
VOCAB = 1_000_000
B = 4096
P = 200

VB = 2048
NBLK = -(-((VOCAB + 1) // 2) // VB)
H = NBLK * VB


def _pack_body(wl_ref, wr_ref, out_ref):
    out_ref[:, 0:64] = wl_ref[...].T
    out_ref[:, 64:128] = wr_ref[...].T


def _pack_table(W_E):
    return pl.pallas_call(
        _pack_body,
        grid=(NBLK,),
        in_specs=[
            pl.BlockSpec((D_MODEL, VB), lambda i: (0, i)),
            pl.BlockSpec(
                (D_MODEL, VB),
                lambda i: (0, jnp.minimum(i + NBLK, (VOCAB - 1) // VB)),
            ),
        ],
        out_specs=pl.BlockSpec((VB, 2 * D_MODEL), lambda i: (i, 0)),
        out_shape=jax.ShapeDtypeStruct((H, 2 * D_MODEL), jnp.float32),
    )(W_E, W_E)


B_TOTAL = B * P
NC, NS = 2, 16
NW = NC * NS
B_PER_W = B_TOTAL // NW
CH = 128
NCHUNK = B_PER_W // CH


def _sc_gather(table, idx):
    mesh = plsc.VectorSubcoreMesh(core_axis_name="c", subcore_axis_name="s")

    @functools.partial(
        pl.kernel,
        mesh=mesh,
        out_type=jax.ShapeDtypeStruct((B_TOTAL, D_MODEL), jnp.float32),
        scratch_types=[
            pltpu.VMEM((NCHUNK, CH), jnp.int32),
            pltpu.VMEM((CH, D_MODEL), jnp.float32),
            pltpu.SemaphoreType.DMA,
        ],
        compiler_params=pltpu.CompilerParams(use_tc_tiling_on_sc=False),
    )
    def k(table_hbm, idx_hbm, out_hbm, idx_v, rows_v, sem):
        wid = lax.axis_index("s") * NC + lax.axis_index("c")
        base = wid * B_PER_W
        pltpu.sync_copy(idx_hbm.at[wid], idx_v)

        def body(j, carry):
            pltpu.async_copy(table_hbm.at[idx_v.at[j]], rows_v, sem).wait()
            pltpu.sync_copy(rows_v, out_hbm.at[pl.ds(base + j * CH, CH)])
            return carry

        lax.fori_loop(0, NCHUNK, body, 0, unroll=False)

    return k(table, idx)


B1 = 128
NB3 = B // B1


def _relayout_body(in_ref, out_ref):
    for p2 in range(P // 2):
        x2 = in_ref[:, p2 * 128:(p2 + 1) * 128]
        out_ref[2 * p2] = x2[:, 0:64].T
        out_ref[2 * p2 + 1] = x2[:, 64:128].T


def _relayout(out_lin):
    v = out_lin.reshape(B, P * D_MODEL)
    return pl.pallas_call(
        _relayout_body,
        grid=(NB3,),
        in_specs=[pl.BlockSpec((B1, P * D_MODEL), lambda i: (i, 0))],
        out_specs=pl.BlockSpec((P, D_MODEL, B1), lambda i: (0, 0, i)),
        out_shape=jax.ShapeDtypeStruct((P, D_MODEL, B), jnp.float32),
    )(v)


def kernel(x, W_E):
    table2 = _pack_table(W_E)
    table = table2.reshape(2 * H, D_MODEL)
    xi = x.astype(jnp.int32)
    j = jnp.where(xi < H, 2 * xi, 2 * (xi - H) + 1)
    idx = j.reshape(NW, NCHUNK, CH)
    out_lin = _sc_gather(table, idx)
    g = _relayout(out_lin)
    return jnp.transpose(g, (2, 0, 1))


# speedup vs baseline: 1.7096x; 1.0535x over previous
import functools

import jax
import jax.numpy as jnp
from jax import lax
from jax.experimental import pallas as pl
from jax.experimental.pallas import tpu as pltpu
from jax.experimental.pallas import tpu_sc as plsc

D_MODEL = 64
VOCAB = 1_000_000
B = 4096
P = 200
P2 = P // 2

VB = 2048
NBLK = -(-((VOCAB + 1) // 2) // VB)
H = NBLK * VB


def _pack_body(wl_ref, wr_ref, out_ref):
    out_ref[:, 0:64] = wl_ref[...].T
    out_ref[:, 64:128] = wr_ref[...].T


def _pack_table(W_E):
    return pl.pallas_call(
        _pack_body,
        grid=(NBLK,),
        in_specs=[
            pl.BlockSpec((D_MODEL, VB), lambda i: (0, i)),
            pl.BlockSpec(
                (D_MODEL, VB),
                lambda i: (0, jnp.minimum(i + NBLK, (VOCAB - 1) // VB)),
            ),
        ],
        out_specs=pl.BlockSpec((VB, 2 * D_MODEL), lambda i: (i, 0)),
        out_shape=jax.ShapeDtypeStruct((H, 2 * D_MODEL), jnp.float32),
    )(W_E, W_E)


NC, NS = 2, 16
NW = NC * NS
B_PER_W = B // NW


def _sc_gather(table, idx):
    mesh = plsc.VectorSubcoreMesh(core_axis_name="c", subcore_axis_name="s")

    @functools.partial(
        pl.kernel,
        mesh=mesh,
        out_type=jax.ShapeDtypeStruct((P2, B, 2 * D_MODEL), jnp.float32),
        scratch_types=[
            pltpu.VMEM((B_PER_W, 2, P2), jnp.int32),
            pltpu.VMEM((P2, D_MODEL), jnp.float32),
            pltpu.VMEM((P2, D_MODEL), jnp.float32),
            pltpu.SemaphoreType.DMA,
            pltpu.SemaphoreType.DMA,
        ],
        compiler_params=pltpu.CompilerParams(use_tc_tiling_on_sc=False),
    )
    def k(table_hbm, idx_hbm, out_hbm, idx_v, rows_e, rows_o, sem_e, sem_o):
        wid = lax.axis_index("s") * NC + lax.axis_index("c")
        b0 = wid * B_PER_W
        pltpu.sync_copy(idx_hbm.at[pl.ds(b0, B_PER_W)], idx_v)

        def body(bi, carry):
            ce = pltpu.async_copy(
                table_hbm.at[idx_v.at[bi, 0]], rows_e, sem_e)
            co = pltpu.async_copy(
                table_hbm.at[idx_v.at[bi, 1]], rows_o, sem_o)
            ce.wait()
            co.wait()
            b = b0 + bi
            pltpu.sync_copy(rows_e, out_hbm.at[:, b, pl.ds(0, D_MODEL)])
            pltpu.sync_copy(rows_o, out_hbm.at[:, b, pl.ds(D_MODEL, D_MODEL)])
            return carry

        lax.fori_loop(0, B_PER_W, body, 0, unroll=False)

    return k(table, idx)


B1 = 1024
NB3 = B // B1


def _relayout_body(in_ref, out_ref):
    x2 = in_ref[...]
    out_ref[0] = x2[:, 0:64].T
    out_ref[1] = x2[:, 64:128].T


def _relayout(m3):
    v = m3.reshape(P2 * B, 2 * D_MODEL)
    return pl.pallas_call(
        _relayout_body,
        grid=(P2, NB3),
        in_specs=[pl.BlockSpec((B1, 2 * D_MODEL), lambda p2, i: (p2 * NB3 + i, 0))],
        out_specs=pl.BlockSpec((2, D_MODEL, B1), lambda p2, i: (p2, 0, i)),
        out_shape=jax.ShapeDtypeStruct((P, D_MODEL, B), jnp.float32),
    )(v)


def kernel(x, W_E):
    table2 = _pack_table(W_E)
    table = table2.reshape(2 * H, D_MODEL)
    xi = x.astype(jnp.int32)
    j = jnp.where(xi < H, 2 * xi, 2 * (xi - H) + 1)
    idx = j.reshape(B, P2, 2).transpose(0, 2, 1)
    m3 = _sc_gather(table, idx)
    g = _relayout(m3)
    return jnp.transpose(g, (2, 0, 1))


# speedup vs baseline: 1.9035x; 1.1134x over previous
import functools

import jax
import jax.numpy as jnp
from jax import lax
from jax.experimental import pallas as pl
from jax.experimental.pallas import tpu as pltpu
from jax.experimental.pallas import tpu_sc as plsc

D_MODEL = 64
VOCAB = 1_000_000
B = 4096
P = 200
P2 = P // 2

VB = 2048
NBLK = -(-((VOCAB + 1) // 2) // VB)
H = NBLK * VB


def _pack_body(wl_ref, wr_ref, out_ref):
    out_ref[:, 0:64] = wl_ref[...].T
    out_ref[:, 64:128] = wr_ref[...].T


def _pack_table(W_E):
    return pl.pallas_call(
        _pack_body,
        grid=(NBLK,),
        in_specs=[
            pl.BlockSpec((D_MODEL, VB), lambda i: (0, i)),
            pl.BlockSpec(
                (D_MODEL, VB),
                lambda i: (0, jnp.minimum(i + NBLK, (VOCAB - 1) // VB)),
            ),
        ],
        out_specs=pl.BlockSpec((VB, 2 * D_MODEL), lambda i: (i, 0)),
        out_shape=jax.ShapeDtypeStruct((H, 2 * D_MODEL), jnp.float32),
    )(W_E, W_E)


NC, NS = 2, 16
NW = NC * NS
NHALF = 2
BH = B // NHALF
B_PER_W = BH // NW


def _sc_gather(table, idx):
    mesh = plsc.VectorSubcoreMesh(core_axis_name="c", subcore_axis_name="s")

    @functools.partial(
        pl.kernel,
        mesh=mesh,
        out_type=jax.ShapeDtypeStruct((P2, BH, 2 * D_MODEL), jnp.float32),
        scratch_types=[
            pltpu.VMEM((B_PER_W, 2, P2), jnp.int32),
            pltpu.VMEM((P2, D_MODEL), jnp.float32),
            pltpu.VMEM((P2, D_MODEL), jnp.float32),
            pltpu.SemaphoreType.DMA,
            pltpu.SemaphoreType.DMA,
        ],
        compiler_params=pltpu.CompilerParams(use_tc_tiling_on_sc=False),
    )
    def k(table_hbm, idx_hbm, out_hbm, idx_v, rows_e, rows_o, sem_e, sem_o):
        wid = lax.axis_index("s") * NC + lax.axis_index("c")
        b0 = wid * B_PER_W
        pltpu.sync_copy(idx_hbm.at[pl.ds(b0, B_PER_W)], idx_v)

        def body(bi, carry):
            ce = pltpu.async_copy(
                table_hbm.at[idx_v.at[bi, 0]], rows_e, sem_e)
            co = pltpu.async_copy(
                table_hbm.at[idx_v.at[bi, 1]], rows_o, sem_o)
            ce.wait()
            co.wait()
            b = b0 + bi
            pltpu.sync_copy(rows_e, out_hbm.at[:, b, pl.ds(0, D_MODEL)])
            pltpu.sync_copy(rows_o, out_hbm.at[:, b, pl.ds(D_MODEL, D_MODEL)])
            return carry

        lax.fori_loop(0, B_PER_W, body, 0, unroll=False)

    return k(table, idx)


B1 = 1024
NB3 = BH // B1


def _relayout_body(*refs):
    in_ref, out_ref = refs[-2], refs[-1]
    x2 = in_ref[...]
    out_ref[0] = x2[:, 0:64].T
    out_ref[1] = x2[:, 64:128].T


def _relayout_half(g_prev, m3, h):
    v = m3.reshape(P2 * BH, 2 * D_MODEL)
    in_spec = pl.BlockSpec((B1, 2 * D_MODEL), lambda p2, i: (p2 * NB3 + i, 0))
    if g_prev is None:
        in_specs, args, aliases = [in_spec], (v,), {}
    else:
        in_specs = [pl.BlockSpec(memory_space=pl.ANY), in_spec]
        args, aliases = (g_prev, v), {0: 0}
    return pl.pallas_call(
        _relayout_body,
        grid=(P2, NB3),
        in_specs=in_specs,
        out_specs=pl.BlockSpec(
            (2, D_MODEL, B1), lambda p2, i: (p2, 0, h * NB3 + i)
        ),
        out_shape=jax.ShapeDtypeStruct((P, D_MODEL, B), jnp.float32),
        input_output_aliases=aliases,
    )(*args)


def kernel(x, W_E):
    table2 = _pack_table(W_E)
    table = table2.reshape(2 * H, D_MODEL)
    xi = x.astype(jnp.int32)
    j = jnp.where(xi < H, 2 * xi, 2 * (xi - H) + 1)
    idx = j.reshape(B, P2, 2).transpose(0, 2, 1)
    m3s = [
        _sc_gather(table, idx[h * BH:(h + 1) * BH])
        for h in range(NHALF)
    ]
    g = _relayout_half(None, m3s[0], 0)
    g = _relayout_half(g, m3s[1], 1)
    return jnp.transpose(g, (2, 0, 1))


# speedup vs baseline: 1.9561x; 1.0276x over previous
import functools

import jax
import jax.numpy as jnp
from jax import lax
from jax.experimental import pallas as pl
from jax.experimental.pallas import tpu as pltpu
from jax.experimental.pallas import tpu_sc as plsc

D_MODEL = 64
VOCAB = 1_000_000
B = 4096
P = 200
P2 = P // 2

VB = 2048
NBLK = -(-((VOCAB + 1) // 2) // VB)
H = NBLK * VB


def _pack_body(wl_ref, wr_ref, out_ref):
    out_ref[:, 0:64] = wl_ref[...].T
    out_ref[:, 64:128] = wr_ref[...].T


def _pack_table(W_E):
    return pl.pallas_call(
        _pack_body,
        grid=(NBLK,),
        in_specs=[
            pl.BlockSpec((D_MODEL, VB), lambda i: (0, i)),
            pl.BlockSpec(
                (D_MODEL, VB),
                lambda i: (0, jnp.minimum(i + NBLK, (VOCAB - 1) // VB)),
            ),
        ],
        out_specs=pl.BlockSpec((VB, 2 * D_MODEL), lambda i: (i, 0)),
        out_shape=jax.ShapeDtypeStruct((H, 2 * D_MODEL), jnp.float32),
    )(W_E, W_E)


NC, NS = 2, 16
NW = NC * NS
NHALF = 2
P2H = P2 // NHALF
B_PER_W = B // NW


def _sc_gather(table, idx):
    mesh = plsc.VectorSubcoreMesh(core_axis_name="c", subcore_axis_name="s")

    @functools.partial(
        pl.kernel,
        mesh=mesh,
        out_type=jax.ShapeDtypeStruct((P2H, B, 2 * D_MODEL), jnp.float32),
        scratch_types=[
            pltpu.VMEM((B_PER_W, 2, P2H), jnp.int32),
            pltpu.VMEM((P2H, D_MODEL), jnp.float32),
            pltpu.VMEM((P2H, D_MODEL), jnp.float32),
            pltpu.SemaphoreType.DMA,
            pltpu.SemaphoreType.DMA,
        ],
        compiler_params=pltpu.CompilerParams(use_tc_tiling_on_sc=False),
    )
    def k(table_hbm, idx_hbm, out_hbm, idx_v, rows_e, rows_o, sem_e, sem_o):
        wid = lax.axis_index("s") * NC + lax.axis_index("c")
        b0 = wid * B_PER_W
        pltpu.sync_copy(idx_hbm.at[pl.ds(b0, B_PER_W)], idx_v)

        def body(bi, carry):
            ce = pltpu.async_copy(
                table_hbm.at[idx_v.at[bi, 0]], rows_e, sem_e)
            co = pltpu.async_copy(
                table_hbm.at[idx_v.at[bi, 1]], rows_o, sem_o)
            ce.wait()
            co.wait()
            b = b0 + bi
            pltpu.sync_copy(rows_e, out_hbm.at[:, b, pl.ds(0, D_MODEL)])
            pltpu.sync_copy(rows_o, out_hbm.at[:, b, pl.ds(D_MODEL, D_MODEL)])
            return carry

        lax.fori_loop(0, B_PER_W, body, 0, unroll=False)

    return k(table, idx)


def _relayout_body(*refs):
    in_ref, out_ref = refs[-2], refs[-1]
    x2 = in_ref[...]
    out_ref[0] = x2[:, 0:64].T
    out_ref[1] = x2[:, 64:128].T


def _relayout_half(g_prev, m3, h):
    v = m3.reshape(P2H * B, 2 * D_MODEL)
    in_spec = pl.BlockSpec((B, 2 * D_MODEL), lambda p2: (p2, 0))
    if g_prev is None:
        in_specs, args, aliases = [in_spec], (v,), {}
    else:
        in_specs = [pl.BlockSpec(memory_space=pl.ANY), in_spec]
        args, aliases = (g_prev, v), {0: 0}
    return pl.pallas_call(
        _relayout_body,
        grid=(P2H,),
        in_specs=in_specs,
        out_specs=pl.BlockSpec(
            (2, D_MODEL, B), lambda p2: (h * P2H + p2, 0, 0)
        ),
        out_shape=jax.ShapeDtypeStruct((P, D_MODEL, B), jnp.float32),
        input_output_aliases=aliases,
    )(*args)


def kernel(x, W_E):
    table2 = _pack_table(W_E)
    table = table2.reshape(2 * H, D_MODEL)
    xi = x.astype(jnp.int32)
    j = jnp.where(xi < H, 2 * xi, 2 * (xi - H) + 1)
    idx = j.reshape(B, P2, 2).transpose(0, 2, 1)
    m3s = [
        _sc_gather(table, idx[:, :, h * P2H:(h + 1) * P2H])
        for h in range(NHALF)
    ]
    g = _relayout_half(None, m3s[0], 0)
    g = _relayout_half(g, m3s[1], 1)
    return jnp.transpose(g, (2, 0, 1))


# speedup vs baseline: 2.5824x; 1.3202x over previous
import functools

import jax
import jax.numpy as jnp
from jax import lax
from jax.experimental import pallas as pl
from jax.experimental.pallas import tpu as pltpu
from jax.experimental.pallas import tpu_sc as plsc

D_MODEL = 64
VOCAB = 1_000_000
B = 4096
P = 200
P2 = P // 2

VB = 4096
NBLK = -(-((VOCAB + 1) // 2) // VB)
H = NBLK * VB


def _pack_body(wl_ref, wr_ref, out_ref):
    out_ref[:, 0:64] = wl_ref[...].T
    out_ref[:, 64:128] = wr_ref[...].T


def _pack_table(W_E):
    return pl.pallas_call(
        _pack_body,
        grid=(NBLK,),
        in_specs=[
            pl.BlockSpec((D_MODEL, VB), lambda i: (0, i)),
            pl.BlockSpec(
                (D_MODEL, VB),
                lambda i: (0, jnp.minimum(i + NBLK, (VOCAB - 1) // VB)),
            ),
        ],
        out_specs=pl.BlockSpec((VB, 2 * D_MODEL), lambda i: (i, 0)),
        out_shape=jax.ShapeDtypeStruct((H, 2 * D_MODEL), jnp.float32),
    )(W_E, W_E)


NC, NS = 2, 16
NW = NC * NS
NHALF = 2
P2H = P2 // NHALF
B_PER_W = B // NW


def _sc_gather(table, idx):
    mesh = plsc.VectorSubcoreMesh(core_axis_name="c", subcore_axis_name="s")

    @functools.partial(
        pl.kernel,
        mesh=mesh,
        out_type=jax.ShapeDtypeStruct((P2H, B, 2 * D_MODEL), jnp.float32),
        scratch_types=[
            pltpu.VMEM((B_PER_W, 2 * P2H), jnp.int32),
            pltpu.VMEM((2, 2 * P2H, D_MODEL), jnp.float32),
            pltpu.SemaphoreType.DMA,
            pltpu.SemaphoreType.DMA,
        ],
        compiler_params=pltpu.CompilerParams(use_tc_tiling_on_sc=False),
    )
    def k(table_hbm, idx_hbm, out_hbm, idx_v, rows_v, sem_g, sem_w):
        wid = lax.axis_index("s") * NC + lax.axis_index("c")
        b0 = wid * B_PER_W
        pltpu.sync_copy(idx_hbm.at[pl.ds(b0, B_PER_W)], idx_v)

        def gather(bi, buf):
            return pltpu.async_copy(
                table_hbm.at[idx_v.at[bi]], rows_v.at[buf], sem_g)

        def write(bi, buf):
            b = b0 + bi
            pltpu.async_copy(
                rows_v.at[buf, pl.ds(0, P2H)],
                out_hbm.at[:, b, pl.ds(0, D_MODEL)], sem_w)
            pltpu.async_copy(
                rows_v.at[buf, pl.ds(P2H, P2H)],
                out_hbm.at[:, b, pl.ds(D_MODEL, D_MODEL)], sem_w)

        def wait_write_pair():
            pltpu.make_async_copy(
                rows_v.at[0, pl.ds(0, P2H)],
                out_hbm.at[:, 0, pl.ds(0, D_MODEL)], sem_w).wait()
            pltpu.make_async_copy(
                rows_v.at[0, pl.ds(P2H, P2H)],
                out_hbm.at[:, 0, pl.ds(D_MODEL, D_MODEL)], sem_w).wait()

        def wait_gather(bi, buf):
            pltpu.make_async_copy(
                table_hbm.at[idx_v.at[bi]], rows_v.at[buf], sem_g).wait()

        gather(0, 0).wait()
        gather(1, 1)
        write(0, 0)

        def body(bi, carry):
            wait_write_pair()
            gather(bi + 1, (bi + 1) % 2)
            wait_gather(bi, bi % 2)
            write(bi, bi % 2)
            return carry

        lax.fori_loop(1, B_PER_W - 1, body, 0, unroll=False)
        bi = B_PER_W - 1
        wait_write_pair()
        wait_gather(bi, bi % 2)
        write(bi, bi % 2)
        wait_write_pair()

    return k(table, idx)


def _relayout_body(*refs):
    in_ref, out_ref = refs[-2], refs[-1]
    x2 = in_ref[...]
    out_ref[0] = x2[:, 0:64].T
    out_ref[1] = x2[:, 64:128].T


def _relayout_half(g_prev, m3, h):
    v = m3.reshape(P2H * B, 2 * D_MODEL)
    in_spec = pl.BlockSpec((B, 2 * D_MODEL), lambda p2: (p2, 0))
    if g_prev is None:
        in_specs, args, aliases = [in_spec], (v,), {}
    else:
        in_specs = [pl.BlockSpec(memory_space=pl.ANY), in_spec]
        args, aliases = (g_prev, v), {0: 0}
    return pl.pallas_call(
        _relayout_body,
        grid=(P2H,),
        in_specs=in_specs,
        out_specs=pl.BlockSpec(
            (2, D_MODEL, B), lambda p2: (h * P2H + p2, 0, 0)
        ),
        out_shape=jax.ShapeDtypeStruct((P, D_MODEL, B), jnp.float32),
        input_output_aliases=aliases,
    )(*args)


def kernel(x, W_E):
    table2 = _pack_table(W_E)
    table = table2.reshape(2 * H, D_MODEL)
    xi = x.astype(jnp.int32)
    j = jnp.where(xi < H, 2 * xi, 2 * (xi - H) + 1)
    idx = j.reshape(B, P2, 2).transpose(0, 2, 1)
    m3s = [
        _sc_gather(
            table,
            idx[:, :, h * P2H:(h + 1) * P2H].reshape(B, 2 * P2H),
        )
        for h in range(NHALF)
    ]
    g = _relayout_half(None, m3s[0], 0)
    g = _relayout_half(g, m3s[1], 1)
    return jnp.transpose(g, (2, 0, 1))


# speedup vs baseline: 3.2415x; 1.2552x over previous
import functools

import jax
import jax.numpy as jnp
from jax import lax
from jax.experimental import pallas as pl
from jax.experimental.pallas import tpu as pltpu
from jax.experimental.pallas import tpu_sc as plsc

D_MODEL = 64
VOCAB = 1_000_000
B = 4096
P = 200
P2 = P // 2

VB = 4096
NBLK = -(-((VOCAB + 1) // 2) // VB)
H = NBLK * VB


def _pack_body(wl_ref, wr_ref, out_ref):
    out_ref[:, 0:64] = wl_ref[...].T
    out_ref[:, 64:128] = wr_ref[...].T


def _pack_table(W_E):
    return pl.pallas_call(
        _pack_body,
        grid=(NBLK,),
        in_specs=[
            pl.BlockSpec((D_MODEL, VB), lambda i: (0, i)),
            pl.BlockSpec(
                (D_MODEL, VB),
                lambda i: (0, jnp.minimum(i + NBLK, (VOCAB - 1) // VB)),
            ),
        ],
        out_specs=pl.BlockSpec((VB, 2 * D_MODEL), lambda i: (i, 0)),
        out_shape=jax.ShapeDtypeStruct((H, 2 * D_MODEL), jnp.float32),
    )(W_E, W_E)


NC, NS = 2, 16
NW = NC * NS
NHALF = 4
P2H = P2 // NHALF
B_PER_W = B // NW


def _sc_gather(table, idx):
    mesh = plsc.VectorSubcoreMesh(core_axis_name="c", subcore_axis_name="s")

    @functools.partial(
        pl.kernel,
        mesh=mesh,
        out_type=jax.ShapeDtypeStruct((P2H, B, 2 * D_MODEL), jnp.float32),
        scratch_types=[
            pltpu.VMEM((B_PER_W, 2 * P2H), jnp.int32),
            pltpu.VMEM((2, 2 * P2H, D_MODEL), jnp.float32),
            pltpu.SemaphoreType.DMA,
            pltpu.SemaphoreType.DMA,
        ],
        compiler_params=pltpu.CompilerParams(use_tc_tiling_on_sc=False),
    )
    def k(table_hbm, idx_hbm, out_hbm, idx_v, rows_v, sem_g, sem_w):
        wid = lax.axis_index("s") * NC + lax.axis_index("c")
        b0 = wid * B_PER_W
        pltpu.sync_copy(idx_hbm.at[pl.ds(b0, B_PER_W)], idx_v)

        def gather(bi, buf):
            return pltpu.async_copy(
                table_hbm.at[idx_v.at[bi]], rows_v.at[buf], sem_g)

        def write(bi, buf):
            b = b0 + bi
            pltpu.async_copy(
                rows_v.at[buf, pl.ds(0, P2H)],
                out_hbm.at[:, b, pl.ds(0, D_MODEL)], sem_w)
            pltpu.async_copy(
                rows_v.at[buf, pl.ds(P2H, P2H)],
                out_hbm.at[:, b, pl.ds(D_MODEL, D_MODEL)], sem_w)

        def wait_write_pair():
            pltpu.make_async_copy(
                rows_v.at[0, pl.ds(0, P2H)],
                out_hbm.at[:, 0, pl.ds(0, D_MODEL)], sem_w).wait()
            pltpu.make_async_copy(
                rows_v.at[0, pl.ds(P2H, P2H)],
                out_hbm.at[:, 0, pl.ds(D_MODEL, D_MODEL)], sem_w).wait()

        def wait_gather(bi, buf):
            pltpu.make_async_copy(
                table_hbm.at[idx_v.at[bi]], rows_v.at[buf], sem_g).wait()

        gather(0, 0).wait()
        gather(1, 1)
        write(0, 0)

        def body(bi, carry):
            wait_write_pair()
            gather(bi + 1, (bi + 1) % 2)
            wait_gather(bi, bi % 2)
            write(bi, bi % 2)
            return carry

        lax.fori_loop(1, B_PER_W - 1, body, 0, unroll=False)
        bi = B_PER_W - 1
        wait_write_pair()
        wait_gather(bi, bi % 2)
        write(bi, bi % 2)
        wait_write_pair()

    return k(table, idx)


def _relayout_body(*refs):
    in_ref, out_ref = refs[-2], refs[-1]
    x2 = in_ref[...]
    out_ref[0] = x2[:, 0:64].T
    out_ref[1] = x2[:, 64:128].T


def _relayout_half(g_prev, m3, h):
    v = m3.reshape(P2H * B, 2 * D_MODEL)
    in_spec = pl.BlockSpec((B, 2 * D_MODEL), lambda p2: (p2, 0))
    if g_prev is None:
        in_specs, args, aliases = [in_spec], (v,), {}
    else:
        in_specs = [pl.BlockSpec(memory_space=pl.ANY), in_spec]
        args, aliases = (g_prev, v), {0: 0}
    return pl.pallas_call(
        _relayout_body,
        grid=(P2H,),
        in_specs=in_specs,
        out_specs=pl.BlockSpec(
            (2, D_MODEL, B), lambda p2: (h * P2H + p2, 0, 0)
        ),
        out_shape=jax.ShapeDtypeStruct((P, D_MODEL, B), jnp.float32),
        input_output_aliases=aliases,
    )(*args)


def kernel(x, W_E):
    table2 = _pack_table(W_E)
    table = table2.reshape(2 * H, D_MODEL)
    xi = x.astype(jnp.int32)
    j = jnp.where(xi < H, 2 * xi, 2 * (xi - H) + 1)
    idx = j.reshape(B, P2, 2).transpose(0, 2, 1)
    m3s = [
        _sc_gather(
            table,
            idx[:, :, h * P2H:(h + 1) * P2H].reshape(B, 2 * P2H),
        )
        for h in range(NHALF)
    ]
    g = _relayout_half(None, m3s[0], 0)
    g = _relayout_half(g, m3s[1], 1)
    return jnp.transpose(g, (2, 0, 1))
